# Initial kernel scaffold; baseline (speedup 1.0000x reference)
import jax
import jax.numpy as jnp
from jax.experimental import pallas as pl


def kernel(x):
    raise NotImplementedError("write your pallas kernel here")


# speedup vs baseline: 196.5191x; 196.5191x over previous
import numpy as np
import jax
import jax.numpy as jnp
from jax.experimental import pallas as pl
from jax.experimental.pallas import tpu as pltpu

_H_WEIGHT = 0.8
_P = 5
_D = 6
_PAD = _D + _P // 2
_MAD_SCALE = 0.6744897501960817
_DB2_HI = np.array([-0.48296291314469025, 0.8365163037378079,
                    -0.2241438680420134, -0.12940952255126037],
                   dtype=np.float32)


def _gauss_taps():
    a = (_P - 1) / 4.0
    u = np.arange(_P, dtype=np.float64) - (_P - 1) / 2.0
    g = np.exp(-(u * u) / (2.0 * a * a))
    g = g / g.sum()
    return g.astype(np.float32)


def _sigma_kernel(ph_ref, med_ref, hh_ref):
    f = _DB2_HI
    C = hh_ref.shape[0]
    N = hh_ref.shape[1]
    n_tot = N * N
    k_rank = float(n_tot // 2)

    for c in range(C):
        acc = None
        for pk in range(2):
            for pc in range(2):
                for kr in range(2):
                    for kc in range(2):
                        coef = float(f[2 * kr + pk]) * float(f[2 * kc + pc])
                        term = ph_ref[2 * pk + pc, c, kr:kr + N, kc:kc + N]
                        acc = term * coef if acc is None else acc + term * coef
        hh_ref[c] = jnp.abs(acc)

    meds = []
    for c in range(C):
        def body(it, lohi, c=c):
            lo, hi = lohi
            mid = 0.5 * (lo + hi)
            cnt = jnp.sum(jnp.where(hh_ref[c] <= mid, 1.0, 0.0))
            big = cnt >= k_rank
            return (jnp.where(big, lo, mid), jnp.where(big, mid, hi))
        lo, hi = jax.lax.fori_loop(
            0, 34, body, (jnp.float32(0.0), jnp.float32(4.0)))
        meds.append(hi)
    med_ref[0, 0] = (meds[0] + meds[1] + meds[2]) / (3.0 * _MAD_SCALE)


def _nlm_kernel(xp_ref, sig_ref, out_ref, wsum_ref, r1_ref, r2_ref):
    i = pl.program_id(0)
    C = out_ref.shape[0]
    BH = out_ref.shape[1]
    W = out_ref.shape[2]
    WD = W + 4
    SH = BH + 16
    y0 = i * BH

    sigma = sig_ref[0, 0]
    var = 2.0 * sigma * sigma
    h = _H_WEIGHT * sigma
    inv_h2 = 1.0 / (h * h)
    g = [float(v) for v in _gauss_taps()]

    out_ref[...] = jnp.zeros_like(out_ref)
    wsum_ref[...] = jnp.zeros_like(wsum_ref)
    for c in range(C):
        slab = xp_ref[c, pl.ds(y0, SH), :]
        r1_ref[c] = slab[6:6 + BH + 4, 6:6 + WD]

    def oy_body(oy, carry):
        up = jnp.where(oy == 0, 0, SH - oy)
        for c in range(C):
            slab = xp_ref[c, pl.ds(y0, SH), :]
            rolled = pltpu.roll(slab, up, axis=0)
            r2_ref[c] = rolled[0:BH + 4, :]
        for ox in range(2 * _D + 1):
            d = None
            for c in range(C):
                dc = r1_ref[c] - r2_ref[c, :, ox:ox + WD]
                dc = dc * dc
                d = dc if d is None else d + dc
            d = d * (1.0 / 3.0)
            t = g[0] * d[0:BH, :]
            for k in range(1, _P):
                t = t + g[k] * d[k:k + BH, :]
            dist = g[0] * t[:, 0:W]
            for k in range(1, _P):
                dist = dist + g[k] * t[:, k:k + W]
            wgt = jnp.exp(jnp.minimum(var - dist, 0.0) * inv_h2)
            wsum_ref[...] += wgt
            for c in range(C):
                ctr = r2_ref[c, 2:2 + BH, ox + 2:ox + 2 + W]
                out_ref[c] += wgt * ctr
        return carry

    jax.lax.fori_loop(0, 2 * _D + 1, oy_body, 0)

    rw = 1.0 / wsum_ref[...]
    for c in range(C):
        out_ref[c] = out_ref[c] * rw


def kernel(x):
    H, W, C = x.shape
    xc = jnp.moveaxis(x, -1, 0)

    xs = jnp.pad(xc, ((0, 0), (3, 3), (3, 3)), mode='symmetric')
    phases = jnp.stack(
        [xs[:, pk::2, pc::2] for pk in range(2) for pc in range(2)])
    S = phases.shape[-1]
    sig = pl.pallas_call(
        _sigma_kernel,
        out_shape=jax.ShapeDtypeStruct((1, 1), jnp.float32),
        out_specs=pl.BlockSpec(memory_space=pltpu.SMEM),
        scratch_shapes=[pltpu.VMEM((C, S - 1, S - 1), jnp.float32)],
        compiler_params=pltpu.CompilerParams(
            vmem_limit_bytes=48 * 1024 * 1024),
        name="nlm_sigma",
    )(phases)

    xp = jnp.pad(xc, ((0, 0), (_PAD, _PAD), (_PAD, _PAD)), mode='reflect')
    BH = 128 if H % 128 == 0 else H
    NB = H // BH
    out = pl.pallas_call(
        _nlm_kernel,
        grid=(NB,),
        in_specs=[
            pl.BlockSpec((C, H + 2 * _PAD, W + 2 * _PAD),
                         lambda i: (0, 0, 0)),
            pl.BlockSpec(memory_space=pltpu.SMEM),
        ],
        out_specs=pl.BlockSpec((C, BH, W), lambda i: (0, i, 0)),
        out_shape=jax.ShapeDtypeStruct((C, H, W), jnp.float32),
        scratch_shapes=[
            pltpu.VMEM((BH, W), jnp.float32),
            pltpu.VMEM((C, BH + 4, W + 4), jnp.float32),
            pltpu.VMEM((C, BH + 4, W + 2 * _PAD), jnp.float32),
        ],
        compiler_params=pltpu.CompilerParams(
            dimension_semantics=("parallel",),
            vmem_limit_bytes=48 * 1024 * 1024),
        name="nlm_main",
    )(xp, sig)
    return jnp.moveaxis(out, 0, -1)


# speedup vs baseline: 282.2156x; 1.4361x over previous
import numpy as np
import jax
import jax.numpy as jnp
from jax.experimental import pallas as pl
from jax.experimental.pallas import tpu as pltpu

_H_WEIGHT = 0.8
_P = 5
_D = 6
_PAD = _D + _P // 2
_MAD_SCALE = 0.6744897501960817
_DB2_HI = np.array([-0.48296291314469025, 0.8365163037378079,
                    -0.2241438680420134, -0.12940952255126037],
                   dtype=np.float32)


def _gauss_taps():
    a = (_P - 1) / 4.0
    u = np.arange(_P, dtype=np.float64) - (_P - 1) / 2.0
    g = np.exp(-(u * u) / (2.0 * a * a))
    g = g / g.sum()
    return g.astype(np.float32)


def _sigma_kernel(ph_ref, med_ref, hh_ref):
    f = _DB2_HI
    C = hh_ref.shape[0]
    N = hh_ref.shape[1]
    n_tot = N * N
    k_rank = float(n_tot // 2)

    for c in range(C):
        acc = None
        for pk in range(2):
            for pc in range(2):
                for kr in range(2):
                    for kc in range(2):
                        coef = float(f[2 * kr + pk]) * float(f[2 * kc + pc])
                        term = ph_ref[2 * pk + pc, c, kr:kr + N, kc:kc + N]
                        acc = term * coef if acc is None else acc + term * coef
        hh_ref[c] = jnp.abs(acc)

    meds = []
    for c in range(C):
        def body(it, lohi, c=c):
            lo, hi = lohi
            mid = 0.5 * (lo + hi)
            cnt = jnp.sum(jnp.where(hh_ref[c] <= mid, 1.0, 0.0))
            big = cnt >= k_rank
            return (jnp.where(big, lo, mid), jnp.where(big, mid, hi))
        lo, hi = jax.lax.fori_loop(
            0, 34, body, (jnp.float32(0.0), jnp.float32(4.0)))
        meds.append(hi)
    med_ref[0, 0] = (meds[0] + meds[1] + meds[2]) / (3.0 * _MAD_SCALE)


def _nlm_kernel(xp_ref, sig_ref, gv_ref, gh_ref, out_ref, wsum_ref,
                r1_ref, r2_ref):
    i = pl.program_id(0)
    C = out_ref.shape[0]
    BH = out_ref.shape[1]
    W = out_ref.shape[2]
    WD = W + 4
    SH = BH + 16
    y0 = i * BH

    sigma = sig_ref[0, 0]
    var = 2.0 * sigma * sigma
    h = _H_WEIGHT * sigma
    inv_h2 = 1.0 / (h * h)

    out_ref[...] = jnp.zeros_like(out_ref)
    wsum_ref[...] = jnp.zeros_like(wsum_ref)
    for c in range(C):
        slab = xp_ref[c, pl.ds(y0, SH), :]
        r1_ref[c] = slab[6:6 + BH + 4, 6:6 + WD]

    def oy_body(oy, carry):
        up = jnp.where(oy == 0, 0, SH - oy)
        for c in range(C):
            slab = xp_ref[c, pl.ds(y0, SH), :]
            rolled = pltpu.roll(slab, up, axis=0)
            r2_ref[c] = rolled[0:BH + 4, :]
        for ox in range(2 * _D + 1):
            d = None
            for c in range(C):
                dc = r1_ref[c] - r2_ref[c, :, ox:ox + WD]
                dc = dc * dc
                d = dc if d is None else d + dc
            t = jnp.dot(gv_ref[...], d.astype(jnp.bfloat16),
                        preferred_element_type=jnp.float32)
            dist = jnp.dot(t.astype(jnp.bfloat16), gh_ref[...],
                           preferred_element_type=jnp.float32)
            wgt = jnp.exp(jnp.minimum(var - dist, 0.0) * inv_h2)
            wsum_ref[...] += wgt
            for c in range(C):
                ctr = r2_ref[c, 2:2 + BH, ox + 2:ox + 2 + W]
                out_ref[c] += wgt * ctr
        return carry

    jax.lax.fori_loop(0, 2 * _D + 1, oy_body, 0)

    rw = 1.0 / wsum_ref[...]
    for c in range(C):
        out_ref[c] = out_ref[c] * rw


def kernel(x):
    H, W, C = x.shape
    xc = jnp.moveaxis(x, -1, 0)

    xs = jnp.pad(xc, ((0, 0), (3, 3), (3, 3)), mode='symmetric')
    phases = jnp.stack(
        [xs[:, pk::2, pc::2] for pk in range(2) for pc in range(2)])
    S = phases.shape[-1]
    sig = pl.pallas_call(
        _sigma_kernel,
        out_shape=jax.ShapeDtypeStruct((1, 1), jnp.float32),
        out_specs=pl.BlockSpec(memory_space=pltpu.SMEM),
        scratch_shapes=[pltpu.VMEM((C, S - 1, S - 1), jnp.float32)],
        compiler_params=pltpu.CompilerParams(
            vmem_limit_bytes=48 * 1024 * 1024),
        name="nlm_sigma",
    )(phases)

    xp = jnp.pad(xc, ((0, 0), (_PAD, _PAD), (_PAD, _PAD)), mode='reflect')
    BH = 128 if H % 128 == 0 else H
    NB = H // BH
    g64 = _gauss_taps().astype(np.float64)
    gv_np = np.zeros((BH, BH + 4), dtype=np.float64)
    for k in range(_P):
        gv_np[np.arange(BH), np.arange(BH) + k] = g64[k] / 3.0
    gh_np = np.zeros((W + 4, W), dtype=np.float64)
    for k in range(_P):
        gh_np[np.arange(W) + k, np.arange(W)] = g64[k]
    gv = jnp.asarray(gv_np.astype(np.float32)).astype(jnp.bfloat16)
    gh = jnp.asarray(gh_np.astype(np.float32)).astype(jnp.bfloat16)
    out = pl.pallas_call(
        _nlm_kernel,
        grid=(NB,),
        in_specs=[
            pl.BlockSpec((C, H + 2 * _PAD, W + 2 * _PAD),
                         lambda i: (0, 0, 0)),
            pl.BlockSpec(memory_space=pltpu.SMEM),
            pl.BlockSpec((BH, BH + 4), lambda i: (0, 0)),
            pl.BlockSpec((W + 4, W), lambda i: (0, 0)),
        ],
        out_specs=pl.BlockSpec((C, BH, W), lambda i: (0, i, 0)),
        out_shape=jax.ShapeDtypeStruct((C, H, W), jnp.float32),
        scratch_shapes=[
            pltpu.VMEM((BH, W), jnp.float32),
            pltpu.VMEM((C, BH + 4, W + 4), jnp.float32),
            pltpu.VMEM((C, BH + 4, W + 2 * _PAD), jnp.float32),
        ],
        compiler_params=pltpu.CompilerParams(
            dimension_semantics=("parallel",),
            vmem_limit_bytes=48 * 1024 * 1024),
        name="nlm_main",
    )(xp, sig, gv, gh)
    return jnp.moveaxis(out, 0, -1)
